# baseline (speedup 1.0000x reference)
import functools

import jax
import jax.numpy as jnp
from jax import lax
from jax.experimental import pallas as pl
from jax.experimental.pallas import tpu as pltpu
from jax.experimental.pallas import tpu_sc as plsc

_NC = 2
_NS = 16
_NW = _NC * _NS

_UNTILED = pltpu.CompilerParams(use_tc_tiling_on_sc=False,
                                needs_layout_passes=False)


def _mesh():
    return plsc.VectorSubcoreMesh(core_axis_name="c", subcore_axis_name="s",
                                  num_cores=_NC)


def _wid():
    return lax.axis_index("s") * _NC + lax.axis_index("c")


def _sc_gather_body(ng, nv, tab16, gidx_hbm, ridx_hbm, cidx_hbm,
                    out_hbm, gidx_v, ridx_v, cidx_v, w_v, e_v, sem):
    w = _wid()
    pltpu.sync_copy(gidx_hbm.at[w], gidx_v)
    pltpu.sync_copy(ridx_hbm.at[w], ridx_v)
    pltpu.sync_copy(cidx_hbm.at[w], cidx_v)
    pltpu.async_copy(tab16.at[gidx_v], w_v, sem).wait()
    for k in range(nv // 16):
        ri = ridx_v[pl.ds(16 * k, 16)]
        ci = cidx_v[pl.ds(16 * k, 16)]
        e_v[pl.ds(16 * k, 16)] = plsc.load_gather(w_v, [ri, ci])
    pltpu.sync_copy(e_v, out_hbm.at[w])


def _sc_gather(table, r):
    n = r.shape[0]
    nwords = table.shape[0] * table.shape[1]
    nrow16 = nwords // 16
    tab16 = table.reshape(nrow16, 16)
    w0 = r * 3
    j = w0 // 16
    s = w0 - j * 16
    gidx = jnp.stack([j, jnp.minimum(j + 1, nrow16 - 1)], axis=-1).reshape(-1)
    pos = s[:, None] + jnp.arange(3, dtype=jnp.int32)[None, :]
    loc = 2 * jnp.arange(n, dtype=jnp.int32)[:, None] + pos // 16
    per_w = n // _NW
    ng = 2 * per_w
    nv = 3 * per_w
    ridx = (loc % ng).reshape(_NW, nv)
    cidx = (pos % 16).reshape(_NW, nv)
    k = pl.kernel(
        functools.partial(_sc_gather_body, ng, nv),
        out_type=jax.ShapeDtypeStruct((_NW, nv), jnp.float32),
        mesh=_mesh(),
        scratch_types=[
            pltpu.VMEM((ng,), jnp.int32),
            pltpu.VMEM((nv,), jnp.int32),
            pltpu.VMEM((nv,), jnp.int32),
            pltpu.VMEM((ng, 16), jnp.float32),
            pltpu.VMEM((nv,), jnp.float32),
            pltpu.SemaphoreType.DMA,
        ],
        compiler_params=_UNTILED,
    )
    return k(tab16, gidx.reshape(_NW, ng), ridx, cidx)


def _dense_body(g_ref, wa_ref, ba_ref, wp_ref, wo_ref, bo_ref, o_ref):
    e = g_ref[...]
    wa = wa_ref[...]
    ba = ba_ref[...]
    wp = wp_ref[...]
    wo = wo_ref[...]
    scores = []
    qs = []
    for (i, j) in ((0, 1), (0, 2), (1, 2)):
        p = e[:, 3 * i:3 * i + 3] * e[:, 3 * j:3 * j + 3]
        h = (p[:, 0:1] * wa[0:1, :] + p[:, 1:2] * wa[1:2, :]
             + p[:, 2:3] * wa[2:3, :] + ba)
        h = jnp.maximum(h, 0.0)
        scores.append(jnp.sum(h * wp, axis=1, keepdims=True))
        qs.append(jnp.sum(p * wo, axis=1, keepdims=True))
    m = jnp.maximum(jnp.maximum(scores[0], scores[1]), scores[2])
    es = [jnp.exp(s - m) for s in scores]
    z = es[0] + es[1] + es[2]
    o_ref[...] = (es[0] * qs[0] + es[1] * qs[1] + es[2] * qs[2]) / z \
        + bo_ref[0, 0]


def _tc_dense(g, W_attn, b_attn, W_proj, W_out, b_out):
    b = g.shape[0]
    s = 2048
    rep = lambda i: (0, 0)
    return pl.pallas_call(
        _dense_body,
        grid=(b // s,),
        in_specs=[
            pl.BlockSpec((s, 9), lambda i: (i, 0)),
            pl.BlockSpec((3, 64), rep),
            pl.BlockSpec((1, 64), rep),
            pl.BlockSpec((1, 64), rep),
            pl.BlockSpec((1, 3), rep),
            pl.BlockSpec((1, 1), rep),
        ],
        out_specs=pl.BlockSpec((s, 1), lambda i: (i, 0)),
        out_shape=jax.ShapeDtypeStruct((b, 1), jnp.float32),
    )(g, W_attn, b_attn.reshape(1, -1), W_proj.reshape(1, -1),
      W_out.reshape(1, -1), b_out.reshape(1, 1))


def kernel(inputs, table, W_attn, b_attn, W_proj, W_out, b_out):
    bsz = inputs.shape[0]
    idx = inputs.astype(jnp.int32).reshape(-1)
    words = _sc_gather(table, idx)
    g = words.reshape(bsz, 9)
    return _tc_dense(g, W_attn, b_attn, W_proj, W_out, b_out)
